# Initial kernel scaffold; baseline (speedup 1.0000x reference)
import functools

import jax
import jax.numpy as jnp
from jax import lax
from jax.experimental import pallas as pl
from jax.experimental.pallas import tpu as pltpu
from jax.experimental.pallas import tpu_sc as plsc

H_IN, W_IN = 480, 854
H_OUT, W_OUT = 800, 1200
WPAD = 864
NCHUNK = WPAD // 16
NW = 32
RPW = H_IN // NW
TOT = 16 * WPAD

_mesh = plsc.VectorSubcoreMesh(core_axis_name="c", subcore_axis_name="s")


@functools.partial(
    pl.kernel,
    out_type=jax.ShapeDtypeStruct((NW, 16), jnp.float32),
    mesh=_mesh,
    scratch_types=[
        pltpu.VMEM((4, 16, WPAD), jnp.float32),
        pltpu.VMEM((TOT,), jnp.float32),
        pltpu.VMEM((TOT,), jnp.float32),
        pltpu.VMEM((WPAD,), jnp.float32),
        pltpu.VMEM((WPAD,), jnp.float32),
        pltpu.VMEM((16,), jnp.float32),
    ],
)
def _sc_any_diff(mask_hbm, scm_hbm, dcm_hbm, out_hbm,
                 raw, mxb, mnb, scv, dcv, accv):
    w = lax.axis_index("s") * 2 + lax.axis_index("c")
    r0 = jnp.minimum(w * RPW, H_IN - 16)
    lr0 = w * RPW - r0

    for b in range(4):
        pltpu.sync_copy(mask_hbm.at[b, pl.ds(r0, 16)], raw.at[b])
    pltpu.sync_copy(scm_hbm, scv)
    pltpu.sync_copy(dcm_hbm, dcv)

    def row_a(lr, _):
        def chunk_a(c, _):
            cc = c * 16
            v0 = raw[0, lr, pl.ds(cc, 16)]
            v1 = raw[1, lr, pl.ds(cc, 16)]
            v2 = raw[2, lr, pl.ds(cc, 16)]
            v3 = raw[3, lr, pl.ds(cc, 16)]
            off = lr * WPAD + cc
            mxb[pl.ds(off, 16)] = jnp.maximum(jnp.maximum(v0, v1),
                                              jnp.maximum(v2, v3))
            mnb[pl.ds(off, 16)] = jnp.minimum(jnp.minimum(v0, v1),
                                              jnp.minimum(v2, v3))
            return 0
        return lax.fori_loop(0, NCHUNK, chunk_a, 0)

    lax.fori_loop(0, 16, row_a, 0)

    iota = lax.iota(jnp.int32, 16)

    def row_b(i, acc_row):
        lr = lr0 + i
        gr = r0 + lr
        dr = lax.rem(gr, 3) == 1
        off_row = lr * WPAD
        offd_row = jnp.where(lr >= 15, off_row, off_row + WPAD)

        def chunk_b(c, acc):
            cc = c * 16
            off = off_row + cc
            offd = offd_row + cc
            a = mxb[pl.ds(off, 16)]
            b = mnb[pl.ds(off, 16)]
            g = a > b
            idx = jnp.minimum(iota + (off + 1), TOT - 1)
            a_s = plsc.load_gather(mxb, [idx])
            b_s = plsc.load_gather(mnb, [idx])
            dc = dcv[pl.ds(cc, 16)] > 0.0
            t2 = jnp.logical_and(dc, (a > b_s) | (a_s > b))
            ad = mxb[pl.ds(offd, 16)]
            bd = mnb[pl.ds(offd, 16)]
            sc = scv[pl.ds(cc, 16)] > 0.0
            t3 = jnp.logical_and(sc, (a > bd) | (ad > b))
            idxd = jnp.minimum(iota + (offd + 1), TOT - 1)
            a_ds = plsc.load_gather(mxb, [idxd])
            b_ds = plsc.load_gather(mnb, [idxd])
            t4 = jnp.logical_and(dc, (a > b_ds) | (a_ds > b))
            hit = g | t2 | jnp.logical_and(t3 | t4, dr)
            return jnp.maximum(acc, jnp.where(hit, 1.0, 0.0))

        return lax.fori_loop(0, NCHUNK, chunk_b, acc_row)

    acc = lax.fori_loop(0, RPW, row_b, jnp.zeros((16,), jnp.float32))
    accv[pl.ds(0, 16)] = acc
    pltpu.sync_copy(accv, out_hbm.at[w])


def _tc_reduce_body(x_ref, o_ref):
    o_ref[...] = jnp.max(x_ref[...], axis=(0, 1), keepdims=True)


_tc_reduce = pl.pallas_call(
    _tc_reduce_body,
    out_shape=jax.ShapeDtypeStruct((1, 1), jnp.float32),
)


def kernel(mask):
    m = mask.reshape(4, H_IN, W_IN)
    mp = jnp.pad(m, ((0, 0), (0, 0), (0, WPAD - W_IN)))

    j = jnp.arange(W_OUT, dtype=jnp.float32)
    x = (j + 0.5) / W_OUT * W_IN - 0.5
    cpj = jnp.clip(jnp.round(x).astype(jnp.int32), 0, W_IN - 1)
    ctj = ((jnp.arange(W_OUT) * W_IN) // W_OUT).astype(jnp.int32)
    sc_mask = jnp.zeros((WPAD,), jnp.float32).at[ctj].max(
        jnp.where(cpj == ctj, 1.0, 0.0))
    dc_mask = jnp.zeros((WPAD,), jnp.float32).at[ctj].max(
        jnp.where(cpj == ctj + 1, 1.0, 0.0))

    partial = _sc_any_diff(mp, sc_mask, dc_mask)
    red = _tc_reduce(partial)
    return red[0, 0] > 0.0


# speedup vs baseline: 1504.5518x; 1504.5518x over previous
import functools

import jax
import jax.numpy as jnp
from jax import lax
from jax.experimental import pallas as pl
from jax.experimental.pallas import tpu as pltpu
from jax.experimental.pallas import tpu_sc as plsc

H_IN, W_IN = 480, 854
H_OUT, W_OUT = 800, 1200
WPAD = 864
NCHUNK = WPAD // 16
NW = 32
RPW = H_IN // NW
STRIP = 16 * WPAD
BATCH_SZ = H_IN * WPAD

_mesh = plsc.VectorSubcoreMesh(core_axis_name="c", subcore_axis_name="s")


@functools.partial(
    pl.kernel,
    out_type=jax.ShapeDtypeStruct((NW * 16,), jnp.float32),
    mesh=_mesh,
    scratch_types=[
        pltpu.VMEM((4 * STRIP,), jnp.float32),
        pltpu.VMEM((STRIP + 16,), jnp.float32),
        pltpu.VMEM((STRIP + 16,), jnp.float32),
        pltpu.VMEM((WPAD,), jnp.float32),
        pltpu.VMEM((WPAD,), jnp.float32),
        pltpu.VMEM((16,), jnp.float32),
    ],
)
def _sc_any_diff(mask_hbm, scm_hbm, dcm_hbm, out_hbm,
                 raw, mxb, mnb, scv, dcv, accv):
    w = lax.axis_index("s") * 2 + lax.axis_index("c")
    r0 = jnp.minimum(w * RPW, H_IN - 16)
    lr0 = w * RPW - r0

    for b in range(4):
        pltpu.sync_copy(mask_hbm.at[pl.ds(b * BATCH_SZ + r0 * WPAD, STRIP)],
                        raw.at[pl.ds(b * STRIP, STRIP)])
    pltpu.sync_copy(scm_hbm, scv)
    pltpu.sync_copy(dcm_hbm, dcv)

    def chunk_a(k, _):
        off = k * 16
        v0 = raw[pl.ds(off, 16)]
        v1 = raw[pl.ds(off + STRIP, 16)]
        v2 = raw[pl.ds(off + 2 * STRIP, 16)]
        v3 = raw[pl.ds(off + 3 * STRIP, 16)]
        mxb[pl.ds(off, 16)] = jnp.maximum(jnp.maximum(v0, v1),
                                          jnp.maximum(v2, v3))
        mnb[pl.ds(off, 16)] = jnp.minimum(jnp.minimum(v0, v1),
                                          jnp.minimum(v2, v3))
        return 0

    lax.fori_loop(0, 16 * NCHUNK, chunk_a, 0)

    zeros16 = jnp.zeros((16,), jnp.float32)
    mxb[pl.ds(STRIP, 16)] = zeros16
    mnb[pl.ds(STRIP, 16)] = zeros16 + 1.0

    def row_b(i, acc_row):
        lr = lr0 + i
        gr = r0 + lr
        rem3 = lax.rem(gr, 3)
        drf = (1 - jnp.abs(rem3 - 1)).astype(jnp.float32)
        off_row = lr * WPAD
        offd_row = jnp.where(lr >= 15, off_row, off_row + WPAD)

        def chunk_b(c, acc):
            cc = c * 16
            off = off_row + cc
            offd = offd_row + cc
            a = mxb[pl.ds(off, 16)]
            b = mnb[pl.ds(off, 16)]
            g = jnp.maximum(a - b, 0.0)
            a_s = mxb[pl.ds(off + 1, 16)]
            b_s = mnb[pl.ds(off + 1, 16)]
            dcf = dcv[pl.ds(cc, 16)]
            t2 = dcf * (jnp.maximum(a - b_s, 0.0) + jnp.maximum(a_s - b, 0.0))
            ad = mxb[pl.ds(offd, 16)]
            bd = mnb[pl.ds(offd, 16)]
            scf = scv[pl.ds(cc, 16)]
            t3 = scf * (jnp.maximum(a - bd, 0.0) + jnp.maximum(ad - b, 0.0))
            a_ds = mxb[pl.ds(offd + 1, 16)]
            b_ds = mnb[pl.ds(offd + 1, 16)]
            t4 = dcf * (jnp.maximum(a - b_ds, 0.0) + jnp.maximum(a_ds - b, 0.0))
            score = g + t2 + drf * (t3 + t4)
            return jnp.maximum(acc, score)

        return lax.fori_loop(0, NCHUNK, chunk_b, acc_row)

    acc = lax.fori_loop(0, RPW, row_b, jnp.zeros((16,), jnp.float32))
    accv[pl.ds(0, 16)] = acc
    pltpu.sync_copy(accv, out_hbm.at[pl.ds(w * 16, 16)])


def _tc_reduce_body(x_ref, o_ref):
    o_ref[...] = jnp.max(x_ref[...], axis=(0, 1), keepdims=True)


_tc_reduce = pl.pallas_call(
    _tc_reduce_body,
    out_shape=jax.ShapeDtypeStruct((1, 1), jnp.float32),
)


def kernel(mask):
    m = mask.reshape(4, H_IN, W_IN)
    mp = jnp.pad(m, ((0, 0), (0, 0), (0, WPAD - W_IN))).reshape(-1)

    j = jnp.arange(W_OUT, dtype=jnp.float32)
    x = (j + 0.5) / W_OUT * W_IN - 0.5
    cpj = jnp.clip(jnp.round(x).astype(jnp.int32), 0, W_IN - 1)
    ctj = ((jnp.arange(W_OUT) * W_IN) // W_OUT).astype(jnp.int32)
    sc_mask = jnp.zeros((WPAD,), jnp.float32).at[ctj].max(
        jnp.where(cpj == ctj, 1.0, 0.0))
    dc_mask = jnp.zeros((WPAD,), jnp.float32).at[ctj].max(
        jnp.where(cpj == ctj + 1, 1.0, 0.0))

    partial = _sc_any_diff(mp, sc_mask, dc_mask)
    red = _tc_reduce(partial.reshape(NW, 16))
    return red[0, 0] > 0.0


# speedup vs baseline: 3837.5392x; 2.5506x over previous
import functools

import jax
import jax.numpy as jnp
import numpy as np
from jax import lax
from jax.experimental import pallas as pl
from jax.experimental.pallas import tpu as pltpu
from jax.experimental.pallas import tpu_sc as plsc

H_IN, W_IN = 480, 854
H_OUT, W_OUT = 800, 1200
WPAD = 864
NCHUNK = WPAD // 16
NW = 32
RPW = H_IN // NW
STRIP = 16 * WPAD
BATCH_SZ = H_IN * WPAD

_mesh = plsc.VectorSubcoreMesh(core_axis_name="c", subcore_axis_name="s")


@functools.partial(
    pl.kernel,
    out_type=jax.ShapeDtypeStruct((NW * 16,), jnp.float32),
    mesh=_mesh,
    scratch_types=[
        pltpu.VMEM((4 * STRIP,), jnp.float32),
        pltpu.VMEM((STRIP + 16,), jnp.float32),
        pltpu.VMEM((STRIP + 16,), jnp.float32),
        pltpu.VMEM((WPAD,), jnp.float32),
        pltpu.VMEM((WPAD,), jnp.float32),
        pltpu.VMEM((16,), jnp.float32),
    ],
)
def _sc_any_diff(mask_hbm, scm_hbm, dcm_hbm, out_hbm,
                 raw, mxb, mnb, scv, dcv, accv):
    w = lax.axis_index("s") * 2 + lax.axis_index("c")
    r0 = jnp.minimum(w * RPW, H_IN - 16)
    lr0 = w * RPW - r0

    for b in range(4):
        pltpu.sync_copy(mask_hbm.at[pl.ds(b * BATCH_SZ + r0 * WPAD, STRIP)],
                        raw.at[pl.ds(b * STRIP, STRIP)])
    pltpu.sync_copy(scm_hbm, scv)
    pltpu.sync_copy(dcm_hbm, dcv)

    def chunk_a(k, _):
        off = k * 16
        v0 = raw[pl.ds(off, 16)]
        v1 = raw[pl.ds(off + STRIP, 16)]
        v2 = raw[pl.ds(off + 2 * STRIP, 16)]
        v3 = raw[pl.ds(off + 3 * STRIP, 16)]
        mxb[pl.ds(off, 16)] = jnp.maximum(jnp.maximum(v0, v1),
                                          jnp.maximum(v2, v3))
        mnb[pl.ds(off, 16)] = jnp.minimum(jnp.minimum(v0, v1),
                                          jnp.minimum(v2, v3))
        return 0

    lax.fori_loop(0, 16 * NCHUNK, chunk_a, 0)

    zeros16 = jnp.zeros((16,), jnp.float32)
    mxb[pl.ds(STRIP, 16)] = zeros16
    mnb[pl.ds(STRIP, 16)] = zeros16 + 1.0

    def row_b(i, acc_row):
        lr = lr0 + i
        gr = r0 + lr
        rem3 = lax.rem(gr, 3)
        drf = (1 - jnp.abs(rem3 - 1)).astype(jnp.float32)
        off_row = lr * WPAD
        offd_row = jnp.where(lr >= 15, off_row, off_row + WPAD)

        def chunk_b(c, acc):
            cc = c * 16
            off = off_row + cc
            offd = offd_row + cc
            a = mxb[pl.ds(off, 16)]
            b = mnb[pl.ds(off, 16)]
            g = jnp.maximum(a - b, 0.0)
            a_s = mxb[pl.ds(off + 1, 16)]
            b_s = mnb[pl.ds(off + 1, 16)]
            dcf = dcv[pl.ds(cc, 16)]
            t2 = dcf * (jnp.maximum(a - b_s, 0.0) + jnp.maximum(a_s - b, 0.0))
            ad = mxb[pl.ds(offd, 16)]
            bd = mnb[pl.ds(offd, 16)]
            scf = scv[pl.ds(cc, 16)]
            t3 = scf * (jnp.maximum(a - bd, 0.0) + jnp.maximum(ad - b, 0.0))
            a_ds = mxb[pl.ds(offd + 1, 16)]
            b_ds = mnb[pl.ds(offd + 1, 16)]
            t4 = dcf * (jnp.maximum(a - b_ds, 0.0) + jnp.maximum(a_ds - b, 0.0))
            score = g + t2 + drf * (t3 + t4)
            return jnp.maximum(acc, score)

        return lax.fori_loop(0, NCHUNK, chunk_b, acc_row)

    acc = lax.fori_loop(0, RPW, row_b, jnp.zeros((16,), jnp.float32))
    accv[pl.ds(0, 16)] = acc
    pltpu.sync_copy(accv, out_hbm.at[pl.ds(w * 16, 16)])


def _tc_reduce_body(x_ref, o_ref):
    o_ref[...] = jnp.max(x_ref[...], axis=(0, 1), keepdims=True)


_tc_reduce = pl.pallas_call(
    _tc_reduce_body,
    out_shape=jax.ShapeDtypeStruct((1, 1), jnp.float32),
)


def _col_masks():
    jj = np.arange(W_OUT)
    ctj = (jj * W_IN) // W_OUT
    cpj = np.clip((W_IN * jj + (W_IN // 2)) // W_OUT, 0, W_IN - 1)
    sc = np.zeros(WPAD, np.float32)
    dc = np.zeros(WPAD, np.float32)
    np.maximum.at(sc, ctj, (cpj == ctj).astype(np.float32))
    np.maximum.at(dc, ctj, (cpj == ctj + 1).astype(np.float32))
    return sc, dc


_SC_MASK, _DC_MASK = _col_masks()


def kernel(mask):
    m = mask.reshape(4, H_IN, W_IN)
    mp = jnp.pad(m, ((0, 0), (0, 0), (0, WPAD - W_IN))).reshape(-1)
    partial = _sc_any_diff(mp, _SC_MASK, _DC_MASK)
    red = _tc_reduce(partial.reshape(NW, 16))
    return red[0, 0] > 0.0


# speedup vs baseline: 4690.2850x; 1.2222x over previous
import functools

import jax
import jax.numpy as jnp
import numpy as np
from jax import lax
from jax.experimental import pallas as pl
from jax.experimental.pallas import tpu as pltpu
from jax.experimental.pallas import tpu_sc as plsc

H_IN, W_IN = 480, 854
H_OUT, W_OUT = 800, 1200
NW = 32
RPW = H_IN // NW
WROWS = 20
WSTART_MAX = H_IN - WROWS
WIN = WROWS * W_IN
NFULL = W_IN // 16
TAILC = W_IN - 16
BATCH_SZ = H_IN * W_IN
MBUF = WROWS * W_IN + 16

_mesh = plsc.VectorSubcoreMesh(core_axis_name="c", subcore_axis_name="s")


@functools.partial(
    pl.kernel,
    out_type=jax.ShapeDtypeStruct((NW * 16,), jnp.float32),
    mesh=_mesh,
    scratch_types=[
        pltpu.VMEM((4 * WIN,), jnp.float32),
        pltpu.VMEM((MBUF,), jnp.float32),
        pltpu.VMEM((MBUF,), jnp.float32),
        pltpu.VMEM((W_IN,), jnp.float32),
        pltpu.VMEM((W_IN,), jnp.float32),
        pltpu.VMEM((16,), jnp.float32),
        pltpu.SemaphoreType.DMA,
    ],
)
def _sc_any_diff(mask_hbm, scm_hbm, dcm_hbm, out_hbm,
                 raw, mxb, mnb, scv, dcv, accv, sem):
    w = lax.axis_index("s") * 2 + lax.axis_index("c")
    row0 = w * RPW
    ws = jnp.minimum((row0 // 4) * 4, WSTART_MAX)
    lr0 = row0 - ws

    copies = [pltpu.async_copy(
        mask_hbm.at[pl.ds(b * BATCH_SZ + ws * W_IN, WIN)],
        raw.at[pl.ds(b * WIN, WIN)], sem) for b in range(4)]
    copies.append(pltpu.async_copy(scm_hbm, scv, sem))
    copies.append(pltpu.async_copy(dcm_hbm, dcv, sem))
    for c in copies:
        c.wait()

    zeros16 = jnp.zeros((16,), jnp.float32)
    mxb[pl.ds((lr0 + 15) * W_IN, 16)] = zeros16
    mnb[pl.ds((lr0 + 15) * W_IN, 16)] = zeros16 + 1.0

    def row_a(lr, _):
        base = lr * W_IN

        def chunk_a(cc, _):
            off = base + cc
            v0 = raw[pl.ds(off, 16)]
            v1 = raw[pl.ds(off + WIN, 16)]
            v2 = raw[pl.ds(off + 2 * WIN, 16)]
            v3 = raw[pl.ds(off + 3 * WIN, 16)]
            mxb[pl.ds(off, 16)] = jnp.maximum(jnp.maximum(v0, v1),
                                              jnp.maximum(v2, v3))
            mnb[pl.ds(off, 16)] = jnp.minimum(jnp.minimum(v0, v1),
                                              jnp.minimum(v2, v3))
            return 0

        lax.fori_loop(0, NFULL, lambda c, x: chunk_a(c * 16, x), 0)
        return chunk_a(TAILC, 0)

    lax.fori_loop(lr0, lr0 + 15, row_a, 0)

    def chunk1(off, cc, acc):
        a = mxb[pl.ds(off, 16)]
        b = mnb[pl.ds(off, 16)]
        a_s = mxb[pl.ds(off + 1, 16)]
        b_s = mnb[pl.ds(off + 1, 16)]
        dcf = dcv[pl.ds(cc, 16)]
        g = jnp.maximum(a - b, 0.0)
        t2 = dcf * (jnp.maximum(a - b_s, 0.0) + jnp.maximum(a_s - b, 0.0))
        return jnp.maximum(acc, g + t2)

    def row1(i, acc_row):
        base = (lr0 + i) * W_IN
        acc_row = lax.fori_loop(
            0, NFULL, lambda c, acc: chunk1(base + c * 16, c * 16, acc),
            acc_row)
        return chunk1(base + TAILC, TAILC, acc_row)

    acc = lax.fori_loop(0, RPW, row1, jnp.zeros((16,), jnp.float32))

    def chunk2(off, cc, acc):
        a = mxb[pl.ds(off, 16)]
        b = mnb[pl.ds(off, 16)]
        ad = mxb[pl.ds(off + W_IN, 16)]
        bd = mnb[pl.ds(off + W_IN, 16)]
        a_ds = mxb[pl.ds(off + W_IN + 1, 16)]
        b_ds = mnb[pl.ds(off + W_IN + 1, 16)]
        scf = scv[pl.ds(cc, 16)]
        dcf = dcv[pl.ds(cc, 16)]
        t3 = scf * (jnp.maximum(a - bd, 0.0) + jnp.maximum(ad - b, 0.0))
        t4 = dcf * (jnp.maximum(a - b_ds, 0.0) + jnp.maximum(a_ds - b, 0.0))
        return jnp.maximum(acc, t3 + t4)

    def row2(k, acc_row):
        base = (lr0 + 1 + 3 * k) * W_IN
        acc_row = lax.fori_loop(
            0, NFULL, lambda c, acc: chunk2(base + c * 16, c * 16, acc),
            acc_row)
        return chunk2(base + TAILC, TAILC, acc_row)

    acc = lax.fori_loop(0, 5, row2, acc)
    accv[pl.ds(0, 16)] = acc
    pltpu.sync_copy(accv, out_hbm.at[pl.ds(w * 16, 16)])


def _tc_reduce_body(x_ref, o_ref):
    o_ref[...] = jnp.max(x_ref[...], axis=(0, 1), keepdims=True)


_tc_reduce = pl.pallas_call(
    _tc_reduce_body,
    out_shape=jax.ShapeDtypeStruct((1, 1), jnp.float32),
)


def _col_masks():
    jj = np.arange(W_OUT)
    ctj = (jj * W_IN) // W_OUT
    cpj = np.clip((W_IN * jj + (W_IN // 2)) // W_OUT, 0, W_IN - 1)
    sc = np.zeros(W_IN, np.float32)
    dc = np.zeros(W_IN, np.float32)
    np.maximum.at(sc, ctj, (cpj == ctj).astype(np.float32))
    np.maximum.at(dc, ctj, (cpj == ctj + 1).astype(np.float32))
    return sc, dc


_SC_MASK, _DC_MASK = _col_masks()


def kernel(mask):
    partial = _sc_any_diff(mask.reshape(-1), _SC_MASK, _DC_MASK)
    red = _tc_reduce(partial.reshape(NW, 16))
    return red[0, 0] > 0.0


# speedup vs baseline: 4720.6480x; 1.0065x over previous
import functools

import jax
import jax.numpy as jnp
import numpy as np
from jax import lax
from jax.experimental import pallas as pl
from jax.experimental.pallas import tpu as pltpu
from jax.experimental.pallas import tpu_sc as plsc

H_IN, W_IN = 480, 854
H_OUT, W_OUT = 800, 1200
NW = 32
RPW = H_IN // NW
WROWS = 20
WSTART_MAX = H_IN - WROWS
WIN = WROWS * W_IN
NQUAD = 13
TAILC0 = W_IN - 32
TAILC = W_IN - 16
BATCH_SZ = H_IN * W_IN
MBUF = WROWS * W_IN + 16

_mesh = plsc.VectorSubcoreMesh(core_axis_name="c", subcore_axis_name="s")


@functools.partial(
    pl.kernel,
    out_type=jax.ShapeDtypeStruct((NW * 16,), jnp.float32),
    mesh=_mesh,
    scratch_types=[
        pltpu.VMEM((4 * WIN,), jnp.float32),
        pltpu.VMEM((MBUF,), jnp.float32),
        pltpu.VMEM((MBUF,), jnp.float32),
        pltpu.VMEM((W_IN,), jnp.float32),
        pltpu.VMEM((W_IN,), jnp.float32),
        pltpu.VMEM((16,), jnp.float32),
        pltpu.SemaphoreType.DMA,
    ],
)
def _sc_any_diff(mask_hbm, scm_hbm, dcm_hbm, out_hbm,
                 raw, mxb, mnb, scv, dcv, accv, sem):
    w = lax.axis_index("s") * 2 + lax.axis_index("c")
    row0 = w * RPW
    ws = jnp.minimum((row0 // 4) * 4, WSTART_MAX)
    lr0 = row0 - ws

    copies = [pltpu.async_copy(
        mask_hbm.at[pl.ds(b * BATCH_SZ + ws * W_IN, WIN)],
        raw.at[pl.ds(b * WIN, WIN)], sem) for b in range(4)]
    copies.append(pltpu.async_copy(scm_hbm, scv, sem))
    copies.append(pltpu.async_copy(dcm_hbm, dcv, sem))
    for c in copies:
        c.wait()

    zeros16 = jnp.zeros((16,), jnp.float32)
    mxb[pl.ds((lr0 + 15) * W_IN, 16)] = zeros16
    mnb[pl.ds((lr0 + 15) * W_IN, 16)] = zeros16 + 1.0

    def row_sweep(base, chunk, acc):
        def quad(c, acc):
            cc = c * 64
            for s in range(0, 64, 16):
                acc = chunk(base + cc + s, cc + s, acc)
            return acc

        acc = lax.fori_loop(0, NQUAD, quad, acc)
        acc = chunk(base + TAILC0, TAILC0, acc)
        return chunk(base + TAILC, TAILC, acc)

    def chunk_a(off, cc, acc):
        v0 = raw[pl.ds(off, 16)]
        v1 = raw[pl.ds(off + WIN, 16)]
        v2 = raw[pl.ds(off + 2 * WIN, 16)]
        v3 = raw[pl.ds(off + 3 * WIN, 16)]
        mxb[pl.ds(off, 16)] = jnp.maximum(jnp.maximum(v0, v1),
                                          jnp.maximum(v2, v3))
        mnb[pl.ds(off, 16)] = jnp.minimum(jnp.minimum(v0, v1),
                                          jnp.minimum(v2, v3))
        return acc

    lax.fori_loop(lr0, lr0 + 15,
                  lambda lr, x: row_sweep(lr * W_IN, chunk_a, x), 0)

    def chunk1(off, cc, acc):
        a = mxb[pl.ds(off, 16)]
        b = mnb[pl.ds(off, 16)]
        a_s = mxb[pl.ds(off + 1, 16)]
        b_s = mnb[pl.ds(off + 1, 16)]
        dcf = dcv[pl.ds(cc, 16)]
        g = jnp.maximum(a - b, 0.0)
        t2 = dcf * (jnp.maximum(a - b_s, 0.0) + jnp.maximum(a_s - b, 0.0))
        return jnp.maximum(acc, g + t2)

    acc = lax.fori_loop(
        0, RPW, lambda i, x: row_sweep((lr0 + i) * W_IN, chunk1, x),
        jnp.zeros((16,), jnp.float32))

    def chunk2(off, cc, acc):
        a = mxb[pl.ds(off, 16)]
        b = mnb[pl.ds(off, 16)]
        ad = mxb[pl.ds(off + W_IN, 16)]
        bd = mnb[pl.ds(off + W_IN, 16)]
        a_ds = mxb[pl.ds(off + W_IN + 1, 16)]
        b_ds = mnb[pl.ds(off + W_IN + 1, 16)]
        scf = scv[pl.ds(cc, 16)]
        dcf = dcv[pl.ds(cc, 16)]
        t3 = scf * (jnp.maximum(a - bd, 0.0) + jnp.maximum(ad - b, 0.0))
        t4 = dcf * (jnp.maximum(a - b_ds, 0.0) + jnp.maximum(a_ds - b, 0.0))
        return jnp.maximum(acc, t3 + t4)

    acc = lax.fori_loop(
        0, 5, lambda k, x: row_sweep((lr0 + 1 + 3 * k) * W_IN, chunk2, x),
        acc)
    accv[pl.ds(0, 16)] = acc
    pltpu.sync_copy(accv, out_hbm.at[pl.ds(w * 16, 16)])


def _tc_reduce_body(x_ref, o_ref):
    o_ref[...] = jnp.max(x_ref[...], axis=(0, 1), keepdims=True)


_tc_reduce = pl.pallas_call(
    _tc_reduce_body,
    out_shape=jax.ShapeDtypeStruct((1, 1), jnp.float32),
)


def _col_masks():
    jj = np.arange(W_OUT)
    ctj = (jj * W_IN) // W_OUT
    cpj = np.clip((W_IN * jj + (W_IN // 2)) // W_OUT, 0, W_IN - 1)
    sc = np.zeros(W_IN, np.float32)
    dc = np.zeros(W_IN, np.float32)
    np.maximum.at(sc, ctj, (cpj == ctj).astype(np.float32))
    np.maximum.at(dc, ctj, (cpj == ctj + 1).astype(np.float32))
    return sc, dc


_SC_MASK, _DC_MASK = _col_masks()


def kernel(mask):
    partial = _sc_any_diff(mask.reshape(-1), _SC_MASK, _DC_MASK)
    red = _tc_reduce(partial.reshape(NW, 16))
    return red[0, 0] > 0.0


# speedup vs baseline: 5726.3307x; 1.2130x over previous
import functools

import jax
import jax.numpy as jnp
import numpy as np
from jax import lax
from jax.experimental import pallas as pl
from jax.experimental.pallas import tpu as pltpu
from jax.experimental.pallas import tpu_sc as plsc

H_IN, W_IN = 480, 854
H_OUT, W_OUT = 800, 1200
NW = 32
RPW = H_IN // NW
WROWS = 24
WSTART_MAX = H_IN - WROWS
NQUAD = 13
TAILC0 = W_IN - 32
TAILC = W_IN - 16
MBUF = 15 * W_IN + 16

_mesh = plsc.VectorSubcoreMesh(core_axis_name="c", subcore_axis_name="s")


@functools.partial(
    pl.kernel,
    out_type=jax.ShapeDtypeStruct((NW * 16,), jnp.float32),
    mesh=_mesh,
    scratch_types=[
        pltpu.VMEM((4, WROWS, W_IN), jnp.float32),
        pltpu.VMEM((MBUF,), jnp.float32),
        pltpu.VMEM((MBUF,), jnp.float32),
        pltpu.VMEM((W_IN,), jnp.float32),
        pltpu.VMEM((W_IN,), jnp.float32),
        pltpu.VMEM((16,), jnp.float32),
        pltpu.SemaphoreType.DMA,
    ],
)
def _sc_any_diff(mask_hbm, scm_hbm, dcm_hbm, out_hbm,
                 raw, mxb, mnb, scv, dcv, accv, sem):
    w = lax.axis_index("s") * 2 + lax.axis_index("c")
    row0 = w * RPW
    ws = jnp.minimum((row0 // 8) * 8, WSTART_MAX)
    lr0 = row0 - ws

    copies = [pltpu.async_copy(
        mask_hbm.at[b, 0, pl.ds(ws, WROWS)], raw.at[b], sem)
        for b in range(4)]
    copies.append(pltpu.async_copy(scm_hbm, scv, sem))
    copies.append(pltpu.async_copy(dcm_hbm, dcv, sem))
    for c in copies:
        c.wait()

    zeros16 = jnp.zeros((16,), jnp.float32)
    mxb[pl.ds(15 * W_IN, 16)] = zeros16
    mnb[pl.ds(15 * W_IN, 16)] = zeros16 + 1.0

    def row_sweep(chunk, acc):
        def quad(c, acc):
            cc = c * 64
            for s in range(0, 64, 16):
                acc = chunk(cc + s, acc)
            return acc

        acc = lax.fori_loop(0, NQUAD, quad, acc)
        acc = chunk(TAILC0, acc)
        return chunk(TAILC, acc)

    def row_a(i, x):
        lr = lr0 + i
        mbase = i * W_IN

        def chunk_a(cc, acc):
            v0 = raw[0, lr, pl.ds(cc, 16)]
            v1 = raw[1, lr, pl.ds(cc, 16)]
            v2 = raw[2, lr, pl.ds(cc, 16)]
            v3 = raw[3, lr, pl.ds(cc, 16)]
            off = mbase + cc
            mxb[pl.ds(off, 16)] = jnp.maximum(jnp.maximum(v0, v1),
                                              jnp.maximum(v2, v3))
            mnb[pl.ds(off, 16)] = jnp.minimum(jnp.minimum(v0, v1),
                                              jnp.minimum(v2, v3))
            return acc

        return row_sweep(chunk_a, x)

    lax.fori_loop(0, RPW, row_a, 0)

    def row1(i, x):
        base = i * W_IN

        def chunk1(cc, acc):
            off = base + cc
            a = mxb[pl.ds(off, 16)]
            b = mnb[pl.ds(off, 16)]
            a_s = mxb[pl.ds(off + 1, 16)]
            b_s = mnb[pl.ds(off + 1, 16)]
            dcf = dcv[pl.ds(cc, 16)]
            g = jnp.maximum(a - b, 0.0)
            t2 = dcf * (jnp.maximum(a - b_s, 0.0) + jnp.maximum(a_s - b, 0.0))
            return jnp.maximum(acc, g + t2)

        return row_sweep(chunk1, x)

    acc = lax.fori_loop(0, RPW, row1, jnp.zeros((16,), jnp.float32))

    def row2(k, x):
        base = (1 + 3 * k) * W_IN

        def chunk2(cc, acc):
            off = base + cc
            a = mxb[pl.ds(off, 16)]
            b = mnb[pl.ds(off, 16)]
            ad = mxb[pl.ds(off + W_IN, 16)]
            bd = mnb[pl.ds(off + W_IN, 16)]
            a_ds = mxb[pl.ds(off + W_IN + 1, 16)]
            b_ds = mnb[pl.ds(off + W_IN + 1, 16)]
            scf = scv[pl.ds(cc, 16)]
            dcf = dcv[pl.ds(cc, 16)]
            t3 = scf * (jnp.maximum(a - bd, 0.0) + jnp.maximum(ad - b, 0.0))
            t4 = dcf * (jnp.maximum(a - b_ds, 0.0) +
                        jnp.maximum(a_ds - b, 0.0))
            return jnp.maximum(acc, t3 + t4)

        return row_sweep(chunk2, x)

    acc = lax.fori_loop(0, 5, row2, acc)
    accv[pl.ds(0, 16)] = acc
    pltpu.sync_copy(accv, out_hbm.at[pl.ds(w * 16, 16)])


def _tc_reduce_body(x_ref, o_ref):
    o_ref[...] = jnp.max(x_ref[...], axis=0, keepdims=True)


_tc_reduce = pl.pallas_call(
    _tc_reduce_body,
    out_shape=jax.ShapeDtypeStruct((1,), jnp.float32),
)


def _col_masks():
    jj = np.arange(W_OUT)
    ctj = (jj * W_IN) // W_OUT
    cpj = np.clip((W_IN * jj + (W_IN // 2)) // W_OUT, 0, W_IN - 1)
    sc = np.zeros(W_IN, np.float32)
    dc = np.zeros(W_IN, np.float32)
    np.maximum.at(sc, ctj, (cpj == ctj).astype(np.float32))
    np.maximum.at(dc, ctj, (cpj == ctj + 1).astype(np.float32))
    return sc, dc


_SC_MASK, _DC_MASK = _col_masks()


def kernel(mask):
    partial = _sc_any_diff(mask, _SC_MASK, _DC_MASK)
    red = _tc_reduce(partial)
    return red[0] > 0.0


# speedup vs baseline: 6022.4130x; 1.0517x over previous
import functools

import jax
import jax.numpy as jnp
import numpy as np
from jax import lax
from jax.experimental import pallas as pl
from jax.experimental.pallas import tpu as pltpu
from jax.experimental.pallas import tpu_sc as plsc

H_IN, W_IN = 480, 854
H_OUT, W_OUT = 800, 1200
NW = 32
RPW = H_IN // NW
WROWS = 24
WSTART_MAX = H_IN - WROWS
NQUAD = 13
TAILC0 = W_IN - 32
TAILC = W_IN - 16
MBUF = 15 * W_IN + 16

_mesh = plsc.VectorSubcoreMesh(core_axis_name="c", subcore_axis_name="s")


@functools.partial(
    pl.kernel,
    out_type=jax.ShapeDtypeStruct((NW * 16,), jnp.float32),
    mesh=_mesh,
    scratch_types=[
        pltpu.VMEM((4, WROWS, W_IN), jnp.float32),
        pltpu.VMEM((MBUF,), jnp.float32),
        pltpu.VMEM((MBUF,), jnp.float32),
        pltpu.VMEM((856,), jnp.float32),
        pltpu.VMEM((856,), jnp.float32),
        pltpu.VMEM((16,), jnp.float32),
        pltpu.SemaphoreType.DMA,
    ],
)
def _sc_any_diff(mask_hbm, colmasks_hbm, out_hbm,
                 raw, mxb, mnb, scv, dcv, accv, sem):
    w = lax.axis_index("s") * 2 + lax.axis_index("c")
    row0 = w * RPW
    ws = jnp.minimum((row0 // 8) * 8, WSTART_MAX)
    lr0 = row0 - ws

    copies = [pltpu.async_copy(
        mask_hbm.at[b, 0, pl.ds(ws, WROWS)], raw.at[b], sem)
        for b in range(4)]
    copies.append(pltpu.async_copy(colmasks_hbm.at[pl.ds(0, 856)], scv, sem))
    copies.append(pltpu.async_copy(colmasks_hbm.at[pl.ds(856, 856)], dcv,
                                   sem))
    for c in copies:
        c.wait()

    zeros16 = jnp.zeros((16,), jnp.float32)

    def init_bound(k, x):
        mxb[pl.ds(k * W_IN, 16)] = zeros16
        mnb[pl.ds(k * W_IN, 16)] = zeros16 + 1.0
        return x

    lax.fori_loop(1, 16, init_bound, 0)

    def rowf(i, x):
        lr = lr0 + i
        mbase = i * W_IN

        def chunkf(cc, acc):
            v0 = raw[0, lr, pl.ds(cc, 16)]
            v1 = raw[1, lr, pl.ds(cc, 16)]
            v2 = raw[2, lr, pl.ds(cc, 16)]
            v3 = raw[3, lr, pl.ds(cc, 16)]
            a = jnp.maximum(jnp.maximum(v0, v1), jnp.maximum(v2, v3))
            b = jnp.minimum(jnp.minimum(v0, v1), jnp.minimum(v2, v3))
            off = mbase + cc
            mxb[pl.ds(off, 16)] = a
            mnb[pl.ds(off, 16)] = b
            a_s = mxb[pl.ds(off + 1, 16)]
            b_s = mnb[pl.ds(off + 1, 16)]
            dcf = dcv[pl.ds(cc, 16)]
            g = jnp.maximum(a - b, 0.0)
            t2 = dcf * jnp.maximum(jnp.maximum(a, a_s) -
                                   jnp.minimum(b, b_s), 0.0)
            return jnp.maximum(acc, g + t2)

        acc = chunkf(TAILC, x)
        acc = chunkf(TAILC0, acc)

        def quad(c, acc):
            cc = (NQUAD - 1 - c) * 64
            for s in (48, 32, 16, 0):
                acc = chunkf(cc + s, acc)
            return acc

        return lax.fori_loop(0, NQUAD, quad, acc)

    acc = lax.fori_loop(0, RPW, rowf, jnp.zeros((16,), jnp.float32))

    def row2(k, x):
        base = (1 + 3 * k) * W_IN

        def chunk2(cc, acc):
            off = base + cc
            a = mxb[pl.ds(off, 16)]
            b = mnb[pl.ds(off, 16)]
            ad = mxb[pl.ds(off + W_IN, 16)]
            bd = mnb[pl.ds(off + W_IN, 16)]
            a_ds = mxb[pl.ds(off + W_IN + 1, 16)]
            b_ds = mnb[pl.ds(off + W_IN + 1, 16)]
            scf = scv[pl.ds(cc, 16)]
            dcf = dcv[pl.ds(cc, 16)]
            t3 = scf * jnp.maximum(jnp.maximum(a, ad) -
                                   jnp.minimum(b, bd), 0.0)
            t4 = dcf * jnp.maximum(jnp.maximum(a, a_ds) -
                                   jnp.minimum(b, b_ds), 0.0)
            return jnp.maximum(acc, t3 + t4)

        def quad(c, acc):
            cc = c * 64
            for s in (0, 16, 32, 48):
                acc = chunk2(cc + s, acc)
            return acc

        acc = lax.fori_loop(0, NQUAD, quad, x)
        acc = chunk2(TAILC0, acc)
        return chunk2(TAILC, acc)

    acc = lax.fori_loop(0, 5, row2, acc)
    accv[pl.ds(0, 16)] = acc
    pltpu.sync_copy(accv, out_hbm.at[pl.ds(w * 16, 16)])


def _tc_reduce_body(x_ref, o_ref):
    o_ref[...] = jnp.max(x_ref[...], axis=0, keepdims=True)


_tc_reduce = pl.pallas_call(
    _tc_reduce_body,
    out_shape=jax.ShapeDtypeStruct((1,), jnp.float32),
)


def _col_masks():
    jj = np.arange(W_OUT)
    ctj = (jj * W_IN) // W_OUT
    cpj = np.clip((W_IN * jj + (W_IN // 2)) // W_OUT, 0, W_IN - 1)
    packed = np.zeros(2 * 856, np.float32)
    np.maximum.at(packed[:W_IN], ctj, (cpj == ctj).astype(np.float32))
    np.maximum.at(packed[856:856 + W_IN], ctj,
                  (cpj == ctj + 1).astype(np.float32))
    return packed


_COL_MASKS = _col_masks()


def kernel(mask):
    partial = _sc_any_diff(mask, _COL_MASKS)
    red = _tc_reduce(partial)
    return red[0] > 0.0


# speedup vs baseline: 6053.2950x; 1.0051x over previous
import functools

import jax
import jax.numpy as jnp
import numpy as np
from jax import lax
from jax.experimental import pallas as pl
from jax.experimental.pallas import tpu as pltpu
from jax.experimental.pallas import tpu_sc as plsc

H_IN, W_IN = 480, 854
H_OUT, W_OUT = 800, 1200
NW = 32
RPW = H_IN // NW
WROWS = 24
WSTART_MAX = H_IN - WROWS
NQUAD = 13
TAILC0 = W_IN - 32
TAILC = W_IN - 16
MBUF = 15 * W_IN + 16

_mesh = plsc.VectorSubcoreMesh(core_axis_name="c", subcore_axis_name="s")


@functools.partial(
    pl.kernel,
    out_type=jax.ShapeDtypeStruct((NW * 16,), jnp.float32),
    mesh=_mesh,
    scratch_types=[
        pltpu.VMEM((4, WROWS, W_IN), jnp.float32),
        pltpu.VMEM((MBUF,), jnp.float32),
        pltpu.VMEM((MBUF,), jnp.float32),
        pltpu.VMEM((856,), jnp.float32),
        pltpu.VMEM((856,), jnp.float32),
        pltpu.VMEM((16,), jnp.float32),
        pltpu.SemaphoreType.DMA,
    ],
)
def _sc_any_diff(mask_hbm, colmasks_hbm, out_hbm,
                 raw, mxb, mnb, scv, dcv, accv, sem):
    w = lax.axis_index("s") * 2 + lax.axis_index("c")
    row0 = w * RPW
    ws = jnp.minimum((row0 // 8) * 8, WSTART_MAX)
    lr0 = row0 - ws

    copies = [pltpu.async_copy(
        mask_hbm.at[b, 0, pl.ds(ws, WROWS)], raw.at[b], sem)
        for b in range(4)]
    copies.append(pltpu.async_copy(colmasks_hbm.at[pl.ds(0, 856)], scv, sem))
    copies.append(pltpu.async_copy(colmasks_hbm.at[pl.ds(856, 856)], dcv,
                                   sem))
    for c in copies:
        c.wait()

    zeros16 = jnp.zeros((16,), jnp.float32)

    def init_bound(k, x):
        mxb[pl.ds(k * W_IN, 16)] = zeros16
        mnb[pl.ds(k * W_IN, 16)] = zeros16 + 1.0
        return x

    lax.fori_loop(1, 16, init_bound, 0)

    def rowf(i, x):
        lr = lr0 + i
        mbase = i * W_IN

        def chunkf(cc, acc):
            v0 = raw[0, lr, pl.ds(cc, 16)]
            v1 = raw[1, lr, pl.ds(cc, 16)]
            v2 = raw[2, lr, pl.ds(cc, 16)]
            v3 = raw[3, lr, pl.ds(cc, 16)]
            a = jnp.maximum(jnp.maximum(v0, v1), jnp.maximum(v2, v3))
            b = jnp.minimum(jnp.minimum(v0, v1), jnp.minimum(v2, v3))
            off = mbase + cc
            mxb[pl.ds(off, 16)] = a
            mnb[pl.ds(off, 16)] = b
            a_s = mxb[pl.ds(off + 1, 16)]
            b_s = mnb[pl.ds(off + 1, 16)]
            dcf = dcv[pl.ds(cc, 16)]
            g = jnp.maximum(a - b, 0.0)
            t2 = dcf * jnp.maximum(jnp.maximum(a, a_s) -
                                   jnp.minimum(b, b_s), 0.0)
            return jnp.maximum(acc, g + t2)

        acc = chunkf(TAILC, x)
        acc = chunkf(TAILC0, acc)

        def pair(c, acc):
            cc = (2 * NQUAD - 1 - c) * 32
            for s in (16, 0):
                acc = chunkf(cc + s, acc)
            return acc

        return lax.fori_loop(0, 2 * NQUAD, pair, acc)

    acc = lax.fori_loop(0, RPW, rowf, jnp.zeros((16,), jnp.float32))

    def row2(k, x):
        base = (1 + 3 * k) * W_IN

        def chunk2(cc, acc):
            off = base + cc
            a = mxb[pl.ds(off, 16)]
            b = mnb[pl.ds(off, 16)]
            ad = mxb[pl.ds(off + W_IN, 16)]
            bd = mnb[pl.ds(off + W_IN, 16)]
            a_ds = mxb[pl.ds(off + W_IN + 1, 16)]
            b_ds = mnb[pl.ds(off + W_IN + 1, 16)]
            scf = scv[pl.ds(cc, 16)]
            dcf = dcv[pl.ds(cc, 16)]
            t3 = scf * jnp.maximum(jnp.maximum(a, ad) -
                                   jnp.minimum(b, bd), 0.0)
            t4 = dcf * jnp.maximum(jnp.maximum(a, a_ds) -
                                   jnp.minimum(b, b_ds), 0.0)
            return jnp.maximum(acc, t3 + t4)

        def pair(c, acc):
            cc = c * 32
            for s in (0, 16):
                acc = chunk2(cc + s, acc)
            return acc

        acc = lax.fori_loop(0, 2 * NQUAD, pair, x)
        acc = chunk2(TAILC0, acc)
        return chunk2(TAILC, acc)

    acc = lax.fori_loop(0, 5, row2, acc)
    accv[pl.ds(0, 16)] = acc
    pltpu.sync_copy(accv, out_hbm.at[pl.ds(w * 16, 16)])


def _tc_reduce_body(x_ref, o_ref):
    o_ref[...] = jnp.max(x_ref[...], axis=0, keepdims=True)


_tc_reduce = pl.pallas_call(
    _tc_reduce_body,
    out_shape=jax.ShapeDtypeStruct((1,), jnp.float32),
)


def _col_masks():
    jj = np.arange(W_OUT)
    ctj = (jj * W_IN) // W_OUT
    cpj = np.clip((W_IN * jj + (W_IN // 2)) // W_OUT, 0, W_IN - 1)
    packed = np.zeros(2 * 856, np.float32)
    np.maximum.at(packed[:W_IN], ctj, (cpj == ctj).astype(np.float32))
    np.maximum.at(packed[856:856 + W_IN], ctj,
                  (cpj == ctj + 1).astype(np.float32))
    return packed


_COL_MASKS = _col_masks()


def kernel(mask):
    partial = _sc_any_diff(mask, _COL_MASKS)
    red = _tc_reduce(partial)
    return red[0] > 0.0
